# Initial kernel scaffold; baseline (speedup 1.0000x reference)
import jax
import jax.numpy as jnp
from jax.experimental import pallas as pl


def kernel(coord, feat, offset, reference_index, params):
    raise NotImplementedError("write your pallas kernel here")


# speedup vs baseline: 2.8170x; 2.8170x over previous
import functools
import jax
import jax.numpy as jnp
from jax import lax
from jax.experimental import pallas as pl
from jax.experimental.pallas import tpu as pltpu
from jax.experimental.pallas import tpu_sc as plsc

EPS = 1e-5
F32 = jnp.float32


def _finalize(s, count, g, be):
    m = s[0] / count
    v = s[1] / count - m * m
    r = lax.rsqrt(v + EPS)
    return jnp.stack([r * g, be - m * r * g])


def _acc(ref, i, y):
    part = jnp.concatenate(
        [jnp.sum(y, axis=0, keepdims=True),
         jnp.sum(y * y, axis=0, keepdims=True)], axis=0)

    @pl.when(i == 0)
    def _():
        ref[...] = jnp.zeros_like(ref)

    ref[...] += part


def _full(shape):
    return pl.BlockSpec(shape, lambda i: tuple(0 for _ in shape))


def _blk(shape):
    return pl.BlockSpec(shape, lambda i: (i,) + tuple(0 for _ in shape[1:]))


def _d0_body(f_ref, w_ref, b_ref, s_ref):
    i = pl.program_id(0)
    y = jnp.dot(f_ref[...], w_ref[...], preferred_element_type=F32) + b_ref[...]
    _acc(s_ref, i, y)


def _d1_body(f_ref, w_ref, b_ref, sb0_ref, fc1_ref, x_ref, s_ref):
    i = pl.program_id(0)
    y0 = jnp.dot(f_ref[...], w_ref[...], preferred_element_type=F32) + b_ref[...]
    x = jax.nn.relu(y0 * sb0_ref[0:1, :] + sb0_ref[1:2, :])
    x_ref[...] = x
    y1 = jnp.dot(x, fc1_ref[...], preferred_element_type=F32)
    _acc(s_ref, i, y1)


def _d2_body(x_ref, fc1_ref, sb1_ref, qw_ref, qb_ref, kw_ref, kb_ref,
             vw_ref, vb_ref, h_ref, v_ref, sq_ref, sk_ref):
    i = pl.program_id(0)
    y1 = jnp.dot(x_ref[...], fc1_ref[...], preferred_element_type=F32)
    h = jax.nn.relu(y1 * sb1_ref[0:1, :] + sb1_ref[1:2, :])
    h_ref[...] = h
    v_ref[...] = jnp.dot(h, vw_ref[...], preferred_element_type=F32) + vb_ref[...]
    yq = jnp.dot(h, qw_ref[...], preferred_element_type=F32) + qb_ref[...]
    yk = jnp.dot(h, kw_ref[...], preferred_element_type=F32) + kb_ref[...]
    _acc(sq_ref, i, yq)
    _acc(sk_ref, i, yk)


def _d3_body(h_ref, qw_ref, qb_ref, sbq_ref, kw_ref, kb_ref, sbk_ref,
             we1_ref, q6_ref, k6_ref):
    h = h_ref[...]
    yq = jnp.dot(h, qw_ref[...], preferred_element_type=F32) + qb_ref[...]
    q = jax.nn.relu(yq * sbq_ref[0:1, :] + sbq_ref[1:2, :])
    q6_ref[...] = jnp.dot(q, we1_ref[...], preferred_element_type=F32)
    yk = jnp.dot(h, kw_ref[...], preferred_element_type=F32) + kb_ref[...]
    k = jax.nn.relu(yk * sbk_ref[0:1, :] + sbk_ref[1:2, :])
    k6_ref[...] = jnp.dot(k, we1_ref[...], preferred_element_type=F32)


def _sc_gather(t1, v, idx, n_workers, per_w, chunk):
    bp = idx.shape[0]
    nit = per_w // chunk
    nsub = chunk // 128
    mesh = plsc.VectorSubcoreMesh(core_axis_name="c", subcore_axis_name="s")

    @functools.partial(
        pl.kernel,
        out_type=[jax.ShapeDtypeStruct((bp, 16), F32),
                  jax.ShapeDtypeStruct((bp, 48), F32)],
        mesh=mesh,
        scratch_types=[pltpu.VMEM((chunk,), jnp.int32),
                       pltpu.VMEM((chunk, 16), F32),
                       pltpu.VMEM((chunk, 48), F32),
                       pltpu.SemaphoreType.DMA],
        compiler_params=pltpu.CompilerParams(use_tc_tiling_on_sc=False),
    )
    def gather(t1_hbm, v_hbm, idx_hbm, t1g_hbm, vg_hbm, idx_v, r16, r48, sem):
        wid = lax.axis_index("s") * 2 + lax.axis_index("c")
        base = wid * per_w

        def body16(it, carry):
            off = base + it * chunk
            pltpu.sync_copy(idx_hbm.at[pl.ds(off, chunk)], idx_v)
            cps = [pltpu.async_copy(t1_hbm.at[idx_v.at[pl.ds(j * 128, 128)]],
                                    r16.at[pl.ds(j * 128, 128)], sem)
                   for j in range(nsub)]
            for c in cps:
                c.wait()
            pltpu.sync_copy(r16, t1g_hbm.at[pl.ds(off, chunk)])
            return carry

        def body48(it, carry):
            off = base + it * chunk
            pltpu.sync_copy(idx_hbm.at[pl.ds(off, chunk)], idx_v)
            cps = [pltpu.async_copy(v_hbm.at[idx_v.at[pl.ds(j * 128, 128)]],
                                    r48.at[pl.ds(j * 128, 128)], sem)
                   for j in range(nsub)]
            for c in cps:
                c.wait()
            pltpu.sync_copy(r48, vg_hbm.at[pl.ds(off, chunk)])
            return carry

        lax.fori_loop(0, nit, body16, 0)
        lax.fori_loop(0, nit, body48, 0)

    return gather(t1, v, idx)


def _pos48(t1g_ref, c16_ref, pb1_ref, pb1b_ref, bpts, k):
    t1g = t1g_ref[...].reshape(bpts, k, 16)
    posf = (t1g[:, :, 0:8] - c16_ref[...][:, None, 0:8]).reshape(bpts * k, 8)
    return jnp.dot(posf, pb1_ref[...], preferred_element_type=F32) + pb1b_ref[...]


def _ea_body(t1g_ref, c16_ref, pb1_ref, pb1b_ref, s_ref, *, bpts, k):
    i = pl.program_id(0)
    y1 = _pos48(t1g_ref, c16_ref, pb1_ref, pb1b_ref, bpts, k)
    _acc(s_ref, i, y1)


def _eb_body(t1g_ref, c16_ref, q6_ref, pb1_ref, pb1b_ref, sbp_ref,
             w26_ref, c0_ref, y2_ref, s_ref, *, bpts, k):
    i = pl.program_id(0)
    y1 = _pos48(t1g_ref, c16_ref, pb1_ref, pb1b_ref, bpts, k)
    a = jax.nn.relu(y1 * sbp_ref[0:1, :] + sbp_ref[1:2, :])
    z = jnp.dot(a, w26_ref[...], preferred_element_type=F32) + c0_ref[...]
    k6g = t1g_ref[...].reshape(bpts, k, 16)[:, :, 8:16]
    y2 = k6g + z.reshape(bpts, k, 8) - q6_ref[...][:, None, :]
    y2f = y2.reshape(bpts * k, 8)
    y2_ref[...] = y2f
    _acc(s_ref, i, y2f)


def _ec_body(t1g_ref, c16_ref, y2_ref, vg_ref, pb1_ref, pb1b_ref, sbp_ref,
             pb2_ref, pb2b_ref, sbw_ref, we2_ref, we2b_ref, e8_ref,
             out_ref, s_ref, *, bpts, k):
    i = pl.program_id(0)
    y1 = _pos48(t1g_ref, c16_ref, pb1_ref, pb1b_ref, bpts, k)
    a = jax.nn.relu(y1 * sbp_ref[0:1, :] + sbp_ref[1:2, :])
    peb = jnp.dot(a, pb2_ref[...], preferred_element_type=F32) + pb2b_ref[...]
    lg = jax.nn.relu(y2_ref[...] * sbw_ref[0:1, :] + sbw_ref[1:2, :])
    lg = jnp.dot(lg, we2_ref[...], preferred_element_type=F32) + we2b_ref[...]
    lg = lg.reshape(bpts, k, 8)
    lg = lg - jnp.max(lg, axis=1, keepdims=True)
    ew = jnp.exp(lg)
    w = ew / jnp.sum(ew, axis=1, keepdims=True)
    w48 = jnp.dot(w.reshape(bpts * k, 8), e8_ref[...],
                  preferred_element_type=F32)
    value = vg_ref[...] + peb
    out = jnp.sum((value * w48).reshape(bpts, k, 48), axis=1)
    out_ref[...] = out
    _acc(s_ref, i, out)


def _ed1_body(g_ref, sb2_ref, fc3_ref, s_ref):
    i = pl.program_id(0)
    h2 = jax.nn.relu(g_ref[...] * sb2_ref[0:1, :] + sb2_ref[1:2, :])
    y3 = jnp.dot(h2, fc3_ref[...], preferred_element_type=F32)
    _acc(s_ref, i, y3)


def _ed2_body(g_ref, x_ref, sb2_ref, fc3_ref, sb3_ref, out_ref):
    h2 = jax.nn.relu(g_ref[...] * sb2_ref[0:1, :] + sb2_ref[1:2, :])
    y3 = jnp.dot(h2, fc3_ref[...], preferred_element_type=F32)
    out_ref[...] = jax.nn.relu(x_ref[...] + y3 * sb3_ref[0:1, :] + sb3_ref[1:2, :])


def _call(body, grid, in_arrs, in_specs, out_shapes, out_specs):
    return pl.pallas_call(
        body, grid=(grid,),
        in_specs=in_specs,
        out_specs=out_specs,
        out_shape=out_shapes,
    )(*in_arrs)


def kernel(coord, feat, offset, reference_index, params):
    P = params
    n = feat.shape[0]
    k = reference_index.shape[1]
    c = P['proj_w'].shape[1]
    nk = n * k

    bpt = 1000
    nb = n // bpt
    bke = bpt * k

    r1 = lambda a: a.reshape(1, -1)
    pad1 = lambda a, w: jnp.pad(a.reshape(1, -1), ((0, 0), (0, w - a.shape[0])))

    s0 = _call(_d0_body, nb,
               [feat, P['proj_w'], r1(P['proj_b'])],
               [_blk((bpt, 6)), _full((6, c)), _full((1, c))],
               jax.ShapeDtypeStruct((2, c), F32), _full((2, c)))
    sb0 = _finalize(s0, n, P['proj_g'], P['proj_be'])

    x, s1 = _call(_d1_body, nb,
                  [feat, P['proj_w'], r1(P['proj_b']), sb0, P['fc1_w']],
                  [_blk((bpt, 6)), _full((6, c)), _full((1, c)),
                   _full((2, c)), _full((c, c))],
                  [jax.ShapeDtypeStruct((n, c), F32),
                   jax.ShapeDtypeStruct((2, c), F32)],
                  [_blk((bpt, c)), _full((2, c))])
    sb1 = _finalize(s1, n, P['bn1_g'], P['bn1_b'])

    h, v, sq, sk = _call(
        _d2_body, nb,
        [x, P['fc1_w'], sb1, P['q_w'], r1(P['q_b']), P['k_w'], r1(P['k_b']),
         P['v_w'], r1(P['v_b'])],
        [_blk((bpt, c)), _full((c, c)), _full((2, c)), _full((c, c)),
         _full((1, c)), _full((c, c)), _full((1, c)), _full((c, c)),
         _full((1, c))],
        [jax.ShapeDtypeStruct((n, c), F32), jax.ShapeDtypeStruct((n, c), F32),
         jax.ShapeDtypeStruct((2, c), F32), jax.ShapeDtypeStruct((2, c), F32)],
        [_blk((bpt, c)), _blk((bpt, c)), _full((2, c)), _full((2, c))])
    sbq = _finalize(sq, n, P['q_g'], P['q_be'])
    sbk = _finalize(sk, n, P['k_g'], P['k_be'])

    we1p = jnp.pad(P['we1_w'], ((0, 0), (0, 2)))
    q6, k6 = _call(
        _d3_body, nb,
        [h, P['q_w'], r1(P['q_b']), sbq, P['k_w'], r1(P['k_b']), sbk, we1p],
        [_blk((bpt, c)), _full((c, c)), _full((1, c)), _full((2, c)),
         _full((c, c)), _full((1, c)), _full((2, c)), _full((c, 8))],
        [jax.ShapeDtypeStruct((n, 8), F32), jax.ShapeDtypeStruct((n, 8), F32)],
        [_blk((bpt, 8)), _blk((bpt, 8))])

    c16 = jnp.pad(coord, ((0, 0), (0, 13)))
    t1 = jnp.concatenate([coord, jnp.zeros((n, 5), F32), k6], axis=1)

    n_workers = 32
    chunk = 1792
    per_w = -(-nk // (n_workers * chunk)) * chunk
    bp = per_w * n_workers
    idx = jnp.pad(reference_index.reshape(-1), (0, bp - nk))
    t1g, vg = _sc_gather(t1, v, idx, n_workers, per_w, chunk)

    pb1p = jnp.pad(P['pb1_w'], ((0, 5), (0, 0)))
    sp = _call(functools.partial(_ea_body, bpts=bpt, k=k), nb,
               [t1g, c16, pb1p, r1(P['pb1_b'])],
               [_blk((bke, 16)), _blk((bpt, 16)), _full((8, c)), _full((1, c))],
               jax.ShapeDtypeStruct((2, c), F32), _full((2, c)))
    sbp = _finalize(sp, nk, P['pb_g'], P['pb_be'])

    w26 = jnp.pad(jnp.dot(P['pb2_w'], P['we1_w']), ((0, 0), (0, 2)))
    c0 = pad1(P['we1_b'] + jnp.dot(P['pb2_b'], P['we1_w']), 8)
    y2, s2 = _call(
        functools.partial(_eb_body, bpts=bpt, k=k), nb,
        [t1g, c16, q6, pb1p, r1(P['pb1_b']), sbp, w26, c0],
        [_blk((bke, 16)), _blk((bpt, 16)), _blk((bpt, 8)), _full((8, c)),
         _full((1, c)), _full((2, c)), _full((c, 8)), _full((1, 8))],
        [jax.ShapeDtypeStruct((nk, 8), F32), jax.ShapeDtypeStruct((2, 8), F32)],
        [_blk((bke, 8)), _full((2, 8))])
    sbw = _finalize(s2, nk, pad1(P['we_g'], 8)[0], pad1(P['we_be'], 8)[0])

    we2p = jnp.pad(P['we2_w'], ((0, 2), (0, 2)))
    we2bp = pad1(P['we2_b'], 8)
    e8 = (jnp.arange(c)[None, :] // (c // 6) ==
          jnp.arange(8)[:, None]).astype(F32)
    gva, sg = _call(
        functools.partial(_ec_body, bpts=bpt, k=k), nb,
        [t1g, c16, y2, vg, pb1p, r1(P['pb1_b']), sbp, P['pb2_w'],
         r1(P['pb2_b']), sbw, we2p, we2bp, e8],
        [_blk((bke, 16)), _blk((bpt, 16)), _blk((bke, 8)), _blk((bke, 48)),
         _full((8, c)), _full((1, c)), _full((2, c)), _full((c, c)),
         _full((1, c)), _full((2, 8)), _full((8, 8)), _full((1, 8)),
         _full((8, c))],
        [jax.ShapeDtypeStruct((n, c), F32), jax.ShapeDtypeStruct((2, c), F32)],
        [_blk((bpt, c)), _full((2, c))])
    sb2 = _finalize(sg, n, P['bn2_g'], P['bn2_b'])

    s3 = _call(_ed1_body, nb,
               [gva, sb2, P['fc3_w']],
               [_blk((bpt, c)), _full((2, c)), _full((c, c))],
               jax.ShapeDtypeStruct((2, c), F32), _full((2, c)))
    sb3 = _finalize(s3, n, P['bn3_g'], P['bn3_b'])

    final = _call(_ed2_body, nb,
                  [gva, x, sb2, P['fc3_w'], sb3],
                  [_blk((bpt, c)), _blk((bpt, c)), _full((2, c)),
                   _full((c, c)), _full((2, c))],
                  jax.ShapeDtypeStruct((n, c), F32), _blk((bpt, c)))
    return final
